# baseline (speedup 1.0000x reference)
import functools

import jax
import jax.numpy as jnp
from jax import lax
from jax.experimental import pallas as pl
from jax.experimental.pallas import tpu as pltpu
from jax.experimental.pallas import tpu_sc as plsc

L = 3
N = 100000
D = 128
E = 16384
H0 = 256
H1 = 128

NW = 32
EPW = E // NW
CH = 128
NCH = EPW // CH
NSUB = CH // 16
NQ = 2 * 2 * L


def _sc_features(z1f, z2f, idxc, wsq):
    mesh = plsc.VectorSubcoreMesh(core_axis_name="c", subcore_axis_name="s")

    @functools.partial(
        pl.kernel,
        mesh=mesh,
        out_type=jax.ShapeDtypeStruct((NW, NQ, EPW), jnp.float32),
        compiler_params=pltpu.CompilerParams(needs_layout_passes=False),
        scratch_types=[
            pltpu.VMEM((L, 2 * EPW), jnp.int32),
            pltpu.VMEM((2, D), jnp.float32),
            pltpu.VMEM((2 * CH, D), jnp.float32),
            pltpu.VMEM((2 * CH, D), jnp.float32),
            pltpu.VMEM((NQ, EPW), jnp.float32),
            pltpu.SemaphoreType.DMA,
            pltpu.SemaphoreType.DMA,
        ],
    )
    def sck(z1_h, z2_h, idxc_h, wsq_h, out_h,
            idxc_v, w_v, bufA, bufB, out_v, semA, semB):
        wid = lax.axis_index("s") * 2 + lax.axis_index("c")
        pltpu.sync_copy(idxc_h.at[wid], idxc_v)
        pltpu.sync_copy(wsq_h, w_v)

        zero = jnp.zeros((16,), jnp.float32)
        iota16 = lax.iota(jnp.int32, 16)
        NU = 2 * L * NCH

        def issue(u, b, sem):
            lc = lax.rem(u, L * NCH)
            l = lc // NCH
            c = lax.rem(lc, NCH)
            src1 = idxc_v.at[l, pl.ds(c * 2 * CH, CH)]
            src2 = idxc_v.at[l, pl.ds(c * 2 * CH + CH, CH)]
            dst1 = b.at[pl.ds(0, CH)]
            dst2 = b.at[pl.ds(CH, CH)]

            @pl.when(u < L * NCH)
            def _():
                pltpu.async_copy(z1_h.at[src1], dst1, sem)
                pltpu.async_copy(z1_h.at[src2], dst2, sem)

            @pl.when(u >= L * NCH)
            def _():
                pltpu.async_copy(z2_h.at[src1], dst1, sem)
                pltpu.async_copy(z2_h.at[src2], dst2, sem)

        def drain(b, sem):
            dummy = z1_h.at[pl.ds(0, CH)]
            pltpu.make_async_copy(dummy, b.at[pl.ds(0, CH)], sem).wait()
            pltpu.make_async_copy(dummy, b.at[pl.ds(CH, CH)], sem).wait()

        def compute(u, b):
            t = u // (L * NCH)
            lc = lax.rem(u, L * NCH)
            l = lc // NCH
            c = lax.rem(lc, NCH)
            q = 2 * (t * L + l)

            def sub_body(sub, _):
                rowi = sub * 16 + iota16
                rowj = rowi + CH

                def dbody(db, accs):
                    wblk = w_v[t, pl.ds(db * 16, 16)]
                    d0, d1, n0, n1, m0, m1 = accs
                    for di in range(16):
                        widx = (iota16 + di) & 15
                        dcol = db * 16 + widx
                        wv = wblk[widx]
                        c1 = plsc.load_gather(b, [rowi, dcol])
                        c2 = plsc.load_gather(b, [rowj, dcol])
                        a1 = wv * c1
                        a2 = wv * c2
                        if di % 2 == 0:
                            d0 += a1 * c2
                            n0 += a1 * c1
                            m0 += a2 * c2
                        else:
                            d1 += a1 * c2
                            n1 += a1 * c1
                            m1 += a2 * c2
                    return (d0, d1, n0, n1, m0, m1)

                d0, d1, n0, n1, m0, m1 = lax.fori_loop(
                    0, D // 16, dbody, (zero,) * 6)
                off = c * CH + sub * 16
                out_v[q, pl.ds(off, 16)] = d0 + d1
                out_v[q + 1, pl.ds(off, 16)] = (n0 + n1) * (m0 + m1)
                return 0

            lax.fori_loop(0, NSUB, sub_body, 0)

        issue(jnp.int32(0), bufA, semA)

        def pair_body(p, _):
            ua = 2 * p
            ub = ua + 1
            issue(ub, bufB, semB)
            drain(bufA, semA)
            compute(ua, bufA)

            @pl.when(ub + 1 < NU)
            def _():
                issue(ub + 1, bufA, semA)

            drain(bufB, semB)
            compute(ub, bufB)
            return 0

        lax.fori_loop(0, NU // 2, pair_body, 0)

        pltpu.sync_copy(out_v, out_h.at[wid])

    return sck(z1f, z2f, idxc, wsq)


def _mlp_body(f_ref, w0_ref, b0_ref, w1_ref, b1_ref, wp_ref, bp_ref, o_ref):
    f = f_ref[0]
    rows = []
    for l in range(L):
        d1 = f[2 * l:2 * l + 1]
        p1 = f[2 * l + 1:2 * l + 2]
        d2 = f[2 * L + 2 * l:2 * L + 2 * l + 1]
        p2 = f[2 * L + 2 * l + 1:2 * L + 2 * l + 2]
        s1 = (d1 / jnp.sqrt(jnp.maximum(p1, 1e-16))
              + d2 / jnp.sqrt(jnp.maximum(p2, 1e-16)))
        s2 = d1 + d2
        rows.append(s1)
        rows.append(s2)
    x = jnp.concatenate(rows, axis=0)
    h = jnp.maximum(
        jnp.dot(w0_ref[...], x, preferred_element_type=jnp.float32)
        + b0_ref[...], 0.0)
    h = jnp.maximum(
        jnp.dot(w1_ref[...], h, preferred_element_type=jnp.float32)
        + b1_ref[...], 0.0)
    o_ref[0] = (jnp.dot(wp_ref[...], h, preferred_element_type=jnp.float32)
                + bp_ref[...])


def _mlp(feat, W0, b0, W1, b1, Wp, bp):
    return pl.pallas_call(
        _mlp_body,
        grid=(NW,),
        in_specs=[
            pl.BlockSpec((1, NQ, EPW), lambda i: (i, 0, 0)),
            pl.BlockSpec((H0, 2 * L), lambda i: (0, 0)),
            pl.BlockSpec((H0, 1), lambda i: (0, 0)),
            pl.BlockSpec((H1, H0), lambda i: (0, 0)),
            pl.BlockSpec((H1, 1), lambda i: (0, 0)),
            pl.BlockSpec((1, H1), lambda i: (0, 0)),
            pl.BlockSpec((1, 1), lambda i: (0, 0)),
        ],
        out_specs=pl.BlockSpec((1, 1, EPW), lambda i: (i, 0, 0)),
        out_shape=jax.ShapeDtypeStruct((NW, 1, EPW), jnp.float32),
    )(feat, W0, b0, W1, b1, Wp, bp)


def kernel(edge_list, z1_trains, z2_trains, weight_vec1, weight_vec2,
           W0, b0, W1, b1, Wp, bp, device):
    z1f = z1_trains.reshape(L * N, D)
    z2f = z2_trains.reshape(L * N, D)
    n1 = edge_list[:, 0].astype(jnp.int32).reshape(NW, 1, EPW)
    n2 = edge_list[:, 1].astype(jnp.int32).reshape(NW, 1, EPW)
    loff = (jnp.arange(L, dtype=jnp.int32) * N).reshape(1, L, 1)
    idx1 = (n1 + loff).reshape(NW, L, NCH, 1, CH)
    idx2 = (n2 + loff).reshape(NW, L, NCH, 1, CH)
    idxc = jnp.concatenate([idx1, idx2], axis=3).reshape(NW, L, 2 * EPW)
    wsq = jnp.concatenate(
        [weight_vec1 * weight_vec1, weight_vec2 * weight_vec2], axis=0)

    feat = _sc_features(z1f, z2f, idxc, wsq)
    pred = _mlp(feat, W0, b0.reshape(H0, 1), W1, b1.reshape(H1, 1),
                Wp, bp.reshape(1, 1))
    return pred.reshape(-1)


# speedup vs baseline: 1.8542x; 1.8542x over previous
import functools

import jax
import jax.numpy as jnp
from jax import lax
from jax.experimental import pallas as pl
from jax.experimental.pallas import tpu as pltpu
from jax.experimental.pallas import tpu_sc as plsc

L = 3
N = 100000
D = 128
E = 16384
H0 = 256
H1 = 128

NW = 32
EPW = E // NW
CH = 128
NCH = EPW // CH
NSUB = CH // 16
NQ = 2 * 2 * L


def _sc_features(z1f, z2f, idxc, wsq):
    mesh = plsc.VectorSubcoreMesh(core_axis_name="c", subcore_axis_name="s")

    @functools.partial(
        pl.kernel,
        mesh=mesh,
        out_type=jax.ShapeDtypeStruct((NW, NQ, EPW), jnp.float32),
        compiler_params=pltpu.CompilerParams(needs_layout_passes=False),
        scratch_types=[
            pltpu.VMEM((L, 2 * EPW), jnp.int32),
            pltpu.VMEM((2, D), jnp.float32),
            pltpu.VMEM((CH, D), jnp.float32),
            pltpu.VMEM((CH, D), jnp.float32),
            pltpu.VMEM((CH, D), jnp.float32),
            pltpu.VMEM((CH, D), jnp.float32),
            pltpu.VMEM((NQ, EPW), jnp.float32),
            pltpu.SemaphoreType.DMA,
            pltpu.SemaphoreType.DMA,
        ],
    )
    def sck(z1_h, z2_h, idxc_h, wsq_h, out_h,
            idxc_v, w_v, bufA1, bufA2, bufB1, bufB2, out_v, semA, semB):
        wid = lax.axis_index("s") * 2 + lax.axis_index("c")
        pltpu.sync_copy(idxc_h.at[wid], idxc_v)
        pltpu.sync_copy(wsq_h, w_v)

        zero = jnp.zeros((16,), jnp.float32)
        iota16 = lax.iota(jnp.int32, 16)
        NU = 2 * L * NCH

        def issue(u, b1, b2, sem):
            lc = lax.rem(u, L * NCH)
            l = lc // NCH
            c = lax.rem(lc, NCH)
            src1 = idxc_v.at[l, pl.ds(c * 2 * CH, CH)]
            src2 = idxc_v.at[l, pl.ds(c * 2 * CH + CH, CH)]

            @pl.when(u < L * NCH)
            def _():
                pltpu.async_copy(z1_h.at[src1], b1, sem)
                pltpu.async_copy(z1_h.at[src2], b2, sem)

            @pl.when(u >= L * NCH)
            def _():
                pltpu.async_copy(z2_h.at[src1], b1, sem)
                pltpu.async_copy(z2_h.at[src2], b2, sem)

        def drain(b1, b2, sem):
            dummy = z1_h.at[pl.ds(0, CH)]
            pltpu.make_async_copy(dummy, b1, sem).wait()
            pltpu.make_async_copy(dummy, b2, sem).wait()

        def compute(u, b1, b2):
            t = u // (L * NCH)
            lc = lax.rem(u, L * NCH)
            l = lc // NCH
            c = lax.rem(lc, NCH)
            q = 2 * (t * L + l)

            def sub_body(sub, _):
                rowi = sub * 16 + iota16

                def dbody(db, accs):
                    wblk = w_v[t, pl.ds(db * 16, 16)]
                    d0, d1, n0, n1, m0, m1 = accs
                    for di in range(16):
                        widx = (iota16 + di) & 15
                        dcol = db * 16 + widx
                        wv = wblk[widx]
                        c1 = plsc.load_gather(b1, [rowi, dcol])
                        c2 = plsc.load_gather(b2, [rowi, dcol])
                        a1 = wv * c1
                        a2 = wv * c2
                        if di % 2 == 0:
                            d0 += a1 * c2
                            n0 += a1 * c1
                            m0 += a2 * c2
                        else:
                            d1 += a1 * c2
                            n1 += a1 * c1
                            m1 += a2 * c2
                    return (d0, d1, n0, n1, m0, m1)

                d0, d1, n0, n1, m0, m1 = lax.fori_loop(
                    0, D // 16, dbody, (zero,) * 6)
                off = c * CH + sub * 16
                out_v[q, pl.ds(off, 16)] = d0 + d1
                out_v[q + 1, pl.ds(off, 16)] = (n0 + n1) * (m0 + m1)
                return 0

            lax.fori_loop(0, NSUB, sub_body, 0)

        issue(jnp.int32(0), bufA1, bufA2, semA)

        def pair_body(p, _):
            ua = 2 * p
            ub = ua + 1
            issue(ub, bufB1, bufB2, semB)
            drain(bufA1, bufA2, semA)
            compute(ua, bufA1, bufA2)

            @pl.when(ub + 1 < NU)
            def _():
                issue(ub + 1, bufA1, bufA2, semA)

            drain(bufB1, bufB2, semB)
            compute(ub, bufB1, bufB2)
            return 0

        lax.fori_loop(0, NU // 2, pair_body, 0)

        pltpu.sync_copy(out_v, out_h.at[wid])

    return sck(z1f, z2f, idxc, wsq)


def _mlp_body(f_ref, w0_ref, b0_ref, w1_ref, b1_ref, wp_ref, bp_ref, o_ref):
    f = f_ref[0]
    rows = []
    for l in range(L):
        d1 = f[2 * l:2 * l + 1]
        p1 = f[2 * l + 1:2 * l + 2]
        d2 = f[2 * L + 2 * l:2 * L + 2 * l + 1]
        p2 = f[2 * L + 2 * l + 1:2 * L + 2 * l + 2]
        s1 = (d1 / jnp.sqrt(jnp.maximum(p1, 1e-16))
              + d2 / jnp.sqrt(jnp.maximum(p2, 1e-16)))
        s2 = d1 + d2
        rows.append(s1)
        rows.append(s2)
    x = jnp.concatenate(rows, axis=0)
    h = jnp.maximum(
        jnp.dot(w0_ref[...], x, preferred_element_type=jnp.float32)
        + b0_ref[...], 0.0)
    h = jnp.maximum(
        jnp.dot(w1_ref[...], h, preferred_element_type=jnp.float32)
        + b1_ref[...], 0.0)
    o_ref[0] = (jnp.dot(wp_ref[...], h, preferred_element_type=jnp.float32)
                + bp_ref[...])


def _mlp(feat, W0, b0, W1, b1, Wp, bp):
    return pl.pallas_call(
        _mlp_body,
        grid=(NW,),
        in_specs=[
            pl.BlockSpec((1, NQ, EPW), lambda i: (i, 0, 0)),
            pl.BlockSpec((H0, 2 * L), lambda i: (0, 0)),
            pl.BlockSpec((H0, 1), lambda i: (0, 0)),
            pl.BlockSpec((H1, H0), lambda i: (0, 0)),
            pl.BlockSpec((H1, 1), lambda i: (0, 0)),
            pl.BlockSpec((1, H1), lambda i: (0, 0)),
            pl.BlockSpec((1, 1), lambda i: (0, 0)),
        ],
        out_specs=pl.BlockSpec((1, 1, EPW), lambda i: (i, 0, 0)),
        out_shape=jax.ShapeDtypeStruct((NW, 1, EPW), jnp.float32),
    )(feat, W0, b0, W1, b1, Wp, bp)


def kernel(edge_list, z1_trains, z2_trains, weight_vec1, weight_vec2,
           W0, b0, W1, b1, Wp, bp, device):
    z1f = z1_trains.reshape(L * N, D)
    z2f = z2_trains.reshape(L * N, D)
    n1 = edge_list[:, 0].astype(jnp.int32).reshape(NW, 1, EPW)
    n2 = edge_list[:, 1].astype(jnp.int32).reshape(NW, 1, EPW)
    loff = (jnp.arange(L, dtype=jnp.int32) * N).reshape(1, L, 1)
    idx1 = (n1 + loff).reshape(NW, L, NCH, 1, CH)
    idx2 = (n2 + loff).reshape(NW, L, NCH, 1, CH)
    idxc = jnp.concatenate([idx1, idx2], axis=3).reshape(NW, L, 2 * EPW)
    wsq = jnp.concatenate(
        [weight_vec1 * weight_vec1, weight_vec2 * weight_vec2], axis=0)

    feat = _sc_features(z1f, z2f, idxc, wsq)
    pred = _mlp(feat, W0, b0.reshape(H0, 1), W1, b1.reshape(H1, 1),
                Wp, bp.reshape(1, 1))
    return pred.reshape(-1)
